# baseline (speedup 1.0000x reference)
import functools
import itertools as it

import jax
import jax.numpy as jnp
from jax import lax
from jax.experimental import pallas as pl
from jax.experimental.pallas import tpu as pltpu
from jax.experimental.pallas import tpu_sc as plsc

_K = 6
_SKIP = 4
_TRANSIENTS = 200
_P = 3

_B, _R, _T = 8, 16, 2048
_NROWS = _B * _R
_TOUT = _T - _TRANSIENTS
_NLIN = _K
_MONOMS = tuple(it.combinations_with_replacement(range(_NLIN), _P))
_NFEAT = 1 + _NLIN + len(_MONOMS)

_NWORKERS = 32
_ROWS_PER_W = _NROWS // _NWORKERS

_BLK = 16
_BLOCKS_PER_CHUNK = 29
_CHUNK_T = _BLOCKS_PER_CHUNK * _BLK
_NCHUNKS = 4
_XPAD = 2064


def _body(x_hbm, out_hbm, xin, stage):
    cid = lax.axis_index("c")
    sid = lax.axis_index("s")
    wid = sid * 2 + cid
    iota = lax.iota(jnp.int32, _BLK)

    def row_body(rr, carry):
        r = wid * _ROWS_PER_W + rr
        b = r // _R
        rsub = r % _R
        pltpu.sync_copy(x_hbm.at[pl.ds(r * _T, _T)], xin.at[pl.ds(0, _T)])

        for c in range(_NCHUNKS):
            def blk(tb, carry2):
                t0 = c * _CHUNK_T + tb * _BLK
                lin = [xin[pl.ds(t0 + 180 + _SKIP * j, _BLK)] for j in range(_NLIN)]
                pairs = {}
                for a in range(_NLIN):
                    for b in range(a, _NLIN):
                        pairs[(a, b)] = lin[a] * lin[b]
                idx_t = iota + tb * _BLK
                ones = jnp.full((_BLK,), 1.0, dtype=jnp.float32)

                def put(f, val):
                    idx_f = jnp.full((_BLK,), f, dtype=jnp.int32)
                    plsc.store_scatter(stage, [idx_t, idx_f], val)

                put(0, ones)
                for j in range(_NLIN):
                    put(1 + j, lin[j])
                for m, (i, j, k) in enumerate(_MONOMS):
                    put(1 + _NLIN + m, pairs[(i, j)] * lin[k])
                return carry2

            lax.fori_loop(0, _BLOCKS_PER_CHUNK, blk, 0)
            n_t = min(_CHUNK_T, _TOUT - c * _CHUNK_T)
            pltpu.sync_copy(
                stage.at[pl.ds(0, n_t)],
                out_hbm.at[b, rsub, pl.ds(c * _CHUNK_T, n_t)],
            )
        return carry

    lax.fori_loop(0, _ROWS_PER_W, row_body, 0)


@functools.partial(jax.jit)
def kernel(X):
    Xf = X.reshape(_NROWS * _T)
    mesh = plsc.VectorSubcoreMesh(core_axis_name="c", subcore_axis_name="s")
    out = pl.kernel(
        _body,
        out_type=jax.ShapeDtypeStruct((_B, _R, _TOUT, _NFEAT), jnp.float32),
        mesh=mesh,
        compiler_params=pltpu.CompilerParams(
            needs_layout_passes=False, use_tc_tiling_on_sc=False),
        scratch_types=[
            pltpu.VMEM((_XPAD,), jnp.float32),
            pltpu.VMEM((_CHUNK_T, _NFEAT), jnp.float32),
        ],
    )(Xf)
    return out


# speedup vs baseline: 1.3433x; 1.3433x over previous
import functools
import itertools as it

import jax
import jax.numpy as jnp
from jax import lax
from jax.experimental import pallas as pl
from jax.experimental.pallas import tpu as pltpu
from jax.experimental.pallas import tpu_sc as plsc

_K = 6
_SKIP = 4
_TRANSIENTS = 200
_P = 3

_B, _R, _T = 8, 16, 2048
_NROWS = _B * _R
_TOUT = _T - _TRANSIENTS
_NLIN = _K
_MONOMS = tuple(it.combinations_with_replacement(range(_NLIN), _P))
_NFEAT = 1 + _NLIN + len(_MONOMS)
_LANES = 128

_NWORKERS = 32
_ROWS_PER_W = _NROWS // _NWORKERS

_BLK = 16
_BLOCKS_PER_CHUNK = 29
_CHUNK_T = _BLOCKS_PER_CHUNK * _BLK
_NCHUNKS = 4
_XPAD = 2064
_STAGE = _CHUNK_T * _LANES
_PAGE = _TOUT * _LANES


def _chunk_words(c):
    n_t = min(_CHUNK_T, _TOUT - c * _CHUNK_T)
    return n_t * _LANES


def _body(x_hbm, out_hbm, xin, s0, s1, sem0, sem1):
    cid = lax.axis_index("c")
    sid = lax.axis_index("s")
    wid = sid * 2 + cid
    iota128 = lax.iota(jnp.int32, _BLK) * _LANES
    stages = (s0, s1)
    sems = (sem0, sem1)

    def row_body(rr, carry):
        r = wid * _ROWS_PER_W + rr
        pltpu.sync_copy(x_hbm.at[pl.ds(r * _T, _T)], xin.at[pl.ds(0, _T)])

        for c in range(_NCHUNKS):
            buf = c % 2
            stage, sem = stages[buf], sems[buf]
            nw = _chunk_words(c)
            prev_nw = _chunk_words(c - 2) if c >= 2 else _chunk_words(c + 2)

            def wait_prev(prev_nw=prev_nw, stage=stage, sem=sem):
                pltpu.make_async_copy(
                    stage.at[pl.ds(0, prev_nw)],
                    out_hbm.at[pl.ds(0, prev_nw)],
                    sem,
                ).wait()

            if c >= 2:
                wait_prev()
            else:
                @pl.when(rr > 0)
                def _():
                    wait_prev()

            def blk(tb, carry2, c=c, stage=stage):
                t0 = c * _CHUNK_T + tb * _BLK
                lin = [xin[pl.ds(t0 + 180 + _SKIP * j, _BLK)] for j in range(_NLIN)]
                pairs = {}
                for a in range(_NLIN):
                    for b in range(a, _NLIN):
                        pairs[(a, b)] = lin[a] * lin[b]
                base = iota128 + tb * (_BLK * _LANES)
                ones = jnp.full((_BLK,), 1.0, dtype=jnp.float32)
                plsc.store_scatter(stage, [base], ones)
                for j in range(_NLIN):
                    plsc.store_scatter(stage, [base + (1 + j)], lin[j])
                for m, (i, j, k) in enumerate(_MONOMS):
                    plsc.store_scatter(stage, [base + (1 + _NLIN + m)],
                                       pairs[(i, j)] * lin[k])
                return carry2

            lax.fori_loop(0, _BLOCKS_PER_CHUNK, blk, 0)
            pltpu.make_async_copy(
                stage.at[pl.ds(0, nw)],
                out_hbm.at[pl.ds(r * _PAGE + c * _STAGE, nw)],
                sem,
            ).start()
        return carry

    lax.fori_loop(0, _ROWS_PER_W, row_body, 0)
    pltpu.make_async_copy(
        s0.at[pl.ds(0, _chunk_words(2))],
        out_hbm.at[pl.ds(0, _chunk_words(2))], sem0).wait()
    pltpu.make_async_copy(
        s1.at[pl.ds(0, _chunk_words(3))],
        out_hbm.at[pl.ds(0, _chunk_words(3))], sem1).wait()


@functools.partial(jax.jit)
def kernel(X):
    Xf = X.reshape(_NROWS * _T)
    mesh = plsc.VectorSubcoreMesh(core_axis_name="c", subcore_axis_name="s")
    out = pl.kernel(
        _body,
        out_type=jax.ShapeDtypeStruct((_NROWS * _PAGE,), jnp.float32),
        mesh=mesh,
        compiler_params=pltpu.CompilerParams(needs_layout_passes=False),
        scratch_types=[
            pltpu.VMEM((_XPAD,), jnp.float32),
            pltpu.VMEM((_STAGE,), jnp.float32),
            pltpu.VMEM((_STAGE,), jnp.float32),
            pltpu.SemaphoreType.DMA,
            pltpu.SemaphoreType.DMA,
        ],
    )(Xf)
    out = out.reshape(_NROWS, _TOUT, _LANES)[:, :, :_NFEAT]
    return out.reshape(_B, _R, _TOUT, _NFEAT)


# speedup vs baseline: 1.7795x; 1.3247x over previous
import functools
import itertools as it

import jax
import jax.numpy as jnp
from jax import lax
from jax.experimental import pallas as pl
from jax.experimental.pallas import tpu as pltpu
from jax.experimental.pallas import tpu_sc as plsc

_K = 6
_SKIP = 4
_TRANSIENTS = 200
_P = 3

_B, _R, _T = 8, 16, 2048
_NROWS = _B * _R
_TOUT = _T - _TRANSIENTS
_NLIN = _K
_MONOMS = tuple(it.combinations_with_replacement(range(_NLIN), _P))
_NFEAT = 1 + _NLIN + len(_MONOMS)
_LANES = 128

_NWORKERS = 32
_ROWS_PER_W = _NROWS // _NWORKERS

_BLK = 16
_BLOCKS_PER_CHUNK = 29
_CHUNK_T = _BLOCKS_PER_CHUNK * _BLK
_NCHUNKS = 4
_XPAD = 2064
_STAGE = _CHUNK_T * _LANES
_PAGE = _TOUT * _LANES


def _chunk_words(c):
    n_t = min(_CHUNK_T, _TOUT - c * _CHUNK_T)
    return n_t * _LANES


def _body(x_hbm, out_hbm, xin, s63, s0, s1, sem0, sem1):
    cid = lax.axis_index("c")
    sid = lax.axis_index("s")
    wid = sid * 2 + cid
    iota63 = lax.iota(jnp.int32, _BLK) * _NFEAT
    stages = (s0, s1)
    sems = (sem0, sem1)

    def row_body(rr, carry):
        r = wid * _ROWS_PER_W + rr
        pltpu.sync_copy(x_hbm.at[pl.ds(r * _T, _T)], xin.at[pl.ds(0, _T)])

        for c in range(_NCHUNKS):
            buf = c % 2
            stage, sem = stages[buf], sems[buf]
            nw = _chunk_words(c)
            prev_nw = _chunk_words(c - 2) if c >= 2 else _chunk_words(c + 2)

            def wait_prev(prev_nw=prev_nw, stage=stage, sem=sem):
                pltpu.make_async_copy(
                    stage.at[pl.ds(0, prev_nw)],
                    out_hbm.at[pl.ds(0, prev_nw)],
                    sem,
                ).wait()

            if c >= 2:
                wait_prev()
            else:
                @pl.when(rr > 0)
                def _():
                    wait_prev()

            def blk(tb, carry2, c=c, stage=stage):
                t0 = c * _CHUNK_T + tb * _BLK
                lin = [xin[pl.ds(t0 + 180 + _SKIP * j, _BLK)] for j in range(_NLIN)]
                pairs = {}
                for a in range(_NLIN):
                    for b in range(a, _NLIN):
                        pairs[(a, b)] = lin[a] * lin[b]
                ones = jnp.full((_BLK,), 1.0, dtype=jnp.float32)
                plsc.store_scatter(s63, [iota63], ones)
                for j in range(_NLIN):
                    plsc.store_scatter(s63, [iota63 + (1 + j)], lin[j])
                for m, (i, j, k) in enumerate(_MONOMS):
                    plsc.store_scatter(s63, [iota63 + (1 + _NLIN + m)],
                                       pairs[(i, j)] * lin[k])
                base = tb * (_BLK * _LANES)
                for tt in range(_BLK):
                    for g in range(4):
                        v = s63[pl.ds(tt * _NFEAT + 16 * g, _BLK)]
                        stage[pl.ds(base + tt * _LANES + 16 * g, _BLK)] = v
                return carry2

            lax.fori_loop(0, _BLOCKS_PER_CHUNK, blk, 0)
            pltpu.make_async_copy(
                stage.at[pl.ds(0, nw)],
                out_hbm.at[pl.ds(r * _PAGE + c * _STAGE, nw)],
                sem,
            ).start()
        return carry

    lax.fori_loop(0, _ROWS_PER_W, row_body, 0)
    pltpu.make_async_copy(
        s0.at[pl.ds(0, _chunk_words(2))],
        out_hbm.at[pl.ds(0, _chunk_words(2))], sem0).wait()
    pltpu.make_async_copy(
        s1.at[pl.ds(0, _chunk_words(3))],
        out_hbm.at[pl.ds(0, _chunk_words(3))], sem1).wait()


@functools.partial(jax.jit)
def kernel(X):
    Xf = X.reshape(_NROWS * _T)
    mesh = plsc.VectorSubcoreMesh(core_axis_name="c", subcore_axis_name="s")
    out = pl.kernel(
        _body,
        out_type=jax.ShapeDtypeStruct((_NROWS * _PAGE,), jnp.float32),
        mesh=mesh,
        compiler_params=pltpu.CompilerParams(needs_layout_passes=False),
        scratch_types=[
            pltpu.VMEM((_XPAD,), jnp.float32),
            pltpu.VMEM((_BLK * _NFEAT + _BLK,), jnp.float32),
            pltpu.VMEM((_STAGE,), jnp.float32),
            pltpu.VMEM((_STAGE,), jnp.float32),
            pltpu.SemaphoreType.DMA,
            pltpu.SemaphoreType.DMA,
        ],
    )(Xf)
    out = out.reshape(_NROWS, _TOUT, _LANES)[:, :, :_NFEAT]
    return out.reshape(_B, _R, _TOUT, _NFEAT)
